# baseline (speedup 1.0000x reference)
import jax
import jax.numpy as jnp
from jax.experimental import pallas as pl

_NA = 50000
_NM = 2000
_DIM = 128


def _proj_kernel(h_ref, w_ref, b_ref, o_ref):
    o_ref[...] = jnp.dot(h_ref[...], w_ref[...],
                         preferred_element_type=jnp.float32) + b_ref[...]


def _segsum_kernel(ids_ref, h_ref, o_ref):
    i = pl.program_id(0)

    @pl.when(i == 0)
    def _():
        o_ref[...] = jnp.zeros_like(o_ref)

    ids = ids_ref[...]
    mol_iota = jax.lax.broadcasted_iota(jnp.int32, (ids.shape[0], _NM), 1)
    onehot = (ids == mol_iota).astype(jnp.float32)
    o_ref[...] += jax.lax.dot_general(
        onehot, h_ref[...], (((0,), (0,)), ((), ())),
        preferred_element_type=jnp.float32)


def _mol_mlp_kernel(hm_ref, w1_ref, b1_ref, w2_ref, b2_ref, o_ref):
    t = jnp.tanh(jnp.dot(hm_ref[...], w1_ref[...],
                         preferred_element_type=jnp.float32) + b1_ref[...])
    o_ref[...] = jnp.dot(t, w2_ref[...],
                         preferred_element_type=jnp.float32) + b2_ref[...]


def _sum2_kernel(a_ref, b_ref, c_ref, d_ref, ok_ref, oe_ref):
    ok_ref[...] = a_ref[...] + b_ref[...]
    oe_ref[...] = c_ref[...] + d_ref[...]


def _sum3_kernel(a_ref, b_ref, c_ref, d_ref, e_ref, f_ref, ok_ref, oe_ref):
    ok_ref[...] = a_ref[...] + b_ref[...] + c_ref[...]
    oe_ref[...] = d_ref[...] + e_ref[...] + f_ref[...]


def _pair_kernel(k0_ref, k1_ref, e0_ref, e1_ref, oeps_ref, osig_ref):
    oeps_ref[...] = jnp.sqrt(jnp.abs(k0_ref[...] * k1_ref[...]))
    osig_ref[...] = e0_ref[...] * e1_ref[...]


def _pad_rows(x, rows):
    return jnp.concatenate(
        [x, jnp.zeros((rows - x.shape[0],) + x.shape[1:], x.dtype)], axis=0)


def _as_blocks(v, n_pad):
    v = jnp.concatenate([v, jnp.zeros((n_pad - v.shape[0],), v.dtype)])
    return v.reshape(n_pad // 128, 128)


def _pad_idx(idx, n_pad):
    return jnp.concatenate(
        [idx, jnp.zeros((n_pad - idx.shape[0],), idx.dtype)])


def _ew_call(body, n_out, inputs, n_pad):
    rows = n_pad // 128
    outs = pl.pallas_call(
        body,
        out_shape=[jax.ShapeDtypeStruct((rows, 128), jnp.float32)] * n_out,
    )(*inputs)
    return [o.reshape(-1) for o in outs]


def kernel(h, bond_idx, angle_idx0, angle_idx1, angle_idx2, torsion_idx2,
           torsion_idx3, mol_ids, one_four_idx, nonbonded_idx,
           atom_W1, atom_b1, atom_W2, atom_b2,
           bond_W1, bond_b1, bond_W2, bond_b2,
           angle_W1, angle_b1, angle_W2, angle_b2,
           torsion_W1, torsion_b1, torsion_W2, torsion_b2,
           angle0_W, angle0_b, mol_W1, mol_b1, mol_W2, mol_b2):
    f32 = jnp.float32
    d = h.shape[1]

    wc_atom = atom_W1 @ atom_W2
    bc_atom = atom_b1 @ atom_W2 + atom_b2
    wc_bond = bond_W1 @ bond_W2
    bc_bond = bond_b1 @ bond_W2 + bond_b2
    wc_tor = torsion_W1 @ torsion_W2
    bc_tor = torsion_b1 @ torsion_W2 + torsion_b2
    wc_ang = angle_W1 @ angle_W2
    bc_ang = angle_b1 @ angle_W2 + angle_b2
    wa = angle0_W[:d]
    wb = angle0_W[d:2 * d]
    wcq = angle0_W[2 * d:]
    w_ang_ac = (wa + wcq) @ wc_ang
    w_ang_b = 2.0 * (wb @ wc_ang)
    c_ang = 2.0 * (angle0_b @ wc_ang) + bc_ang

    wbig = jnp.zeros((d, _DIM), f32)
    wbig = wbig.at[:, 0:2].set(wc_atom)
    wbig = wbig.at[:, 2:4].set(wc_bond)
    wbig = wbig.at[:, 4:6].set(wc_tor)
    wbig = wbig.at[:, 6:8].set(w_ang_ac)
    wbig = wbig.at[:, 8:10].set(w_ang_b)
    bvec = jnp.zeros((_DIM,), f32)
    bvec = bvec.at[0:2].set(bc_atom)
    bvec = bvec.at[2:4].set(0.5 * bc_bond)
    bvec = bvec.at[4:6].set(0.5 * bc_tor)
    bvec = bvec.at[8:10].set(c_ang)

    blk = 2000
    p = pl.pallas_call(
        _proj_kernel,
        grid=(_NA // blk,),
        in_specs=[pl.BlockSpec((blk, d), lambda i: (i, 0)),
                  pl.BlockSpec((d, _DIM), lambda i: (0, 0)),
                  pl.BlockSpec((1, _DIM), lambda i: (0, 0))],
        out_specs=pl.BlockSpec((blk, _DIM), lambda i: (i, 0)),
        out_shape=jax.ShapeDtypeStruct((_NA, _DIM), f32),
    )(h, wbig, bvec[None, :])

    k_atom = p[:, 0]
    eq_atom = p[:, 1]
    pk_bond = p[:, 2]
    pe_bond = p[:, 3]
    pk_tor = p[:, 4]
    pe_tor = p[:, 5]
    pk_ac = p[:, 6]
    pe_ac = p[:, 7]
    pk_b = p[:, 8]
    pe_b = p[:, 9]

    ba = 512
    na_pad = ((_NA + ba - 1) // ba) * ba
    h_pad = _pad_rows(h, na_pad)
    ids_pad = jnp.concatenate(
        [mol_ids.astype(jnp.int32),
         jnp.full((na_pad - _NA,), -1, jnp.int32)]).reshape(na_pad, 1)
    h_mol = pl.pallas_call(
        _segsum_kernel,
        grid=(na_pad // ba,),
        in_specs=[pl.BlockSpec((ba, 1), lambda i: (i, 0)),
                  pl.BlockSpec((ba, d), lambda i: (i, 0))],
        out_specs=pl.BlockSpec((_NM, d), lambda i: (0, 0)),
        out_shape=jax.ShapeDtypeStruct((_NM, d), f32),
    )(ids_pad, h_pad)

    ru = mol_W1.shape[1]
    w2p = jnp.zeros((ru, _DIM), f32).at[:, 0:1].set(mol_W2)
    b2p = jnp.zeros((_DIM,), f32).at[0:1].set(mol_b2)
    u0 = pl.pallas_call(
        _mol_mlp_kernel,
        out_shape=jax.ShapeDtypeStruct((_NM, _DIM), f32),
    )(h_mol, mol_W1, mol_b1[None, :], w2p, b2p[None, :])[:, 0]

    def pad_to(n):
        return ((n + 1023) // 1024) * 1024

    nb_pad = pad_to(bond_idx.shape[1])
    b0 = _pad_idx(bond_idx[0], nb_pad)
    b1 = _pad_idx(bond_idx[1], nb_pad)
    kb, eb = _ew_call(_sum2_kernel, 2, [
        pk_bond[b0].reshape(-1, 128), pk_bond[b1].reshape(-1, 128),
        pe_bond[b0].reshape(-1, 128), pe_bond[b1].reshape(-1, 128)], nb_pad)
    k_bond = kb[:bond_idx.shape[1]]
    eq_bond = eb[:bond_idx.shape[1]]

    nt_pad = pad_to(torsion_idx2.shape[0])
    t2 = _pad_idx(torsion_idx2, nt_pad)
    t3 = _pad_idx(torsion_idx3, nt_pad)
    kt, et = _ew_call(_sum2_kernel, 2, [
        pk_tor[t3].reshape(-1, 128), pk_tor[t2].reshape(-1, 128),
        pe_tor[t3].reshape(-1, 128), pe_tor[t2].reshape(-1, 128)], nt_pad)
    k_torsion = kt[:torsion_idx2.shape[0]]
    eq_torsion = et[:torsion_idx2.shape[0]]

    ng_pad = pad_to(angle_idx0.shape[0])
    a0 = _pad_idx(angle_idx0, ng_pad)
    a1 = _pad_idx(angle_idx1, ng_pad)
    a2 = _pad_idx(angle_idx2, ng_pad)
    ka, ea = _ew_call(_sum3_kernel, 2, [
        pk_ac[a0].reshape(-1, 128), pk_ac[a2].reshape(-1, 128),
        pk_b[a1].reshape(-1, 128),
        pe_ac[a0].reshape(-1, 128), pe_ac[a2].reshape(-1, 128),
        pe_b[a1].reshape(-1, 128)], ng_pad)
    k_angle = ka[:angle_idx0.shape[0]]
    eq_angle = ea[:angle_idx0.shape[0]]

    def pair_terms(idx):
        n = idx.shape[1]
        n_pad = pad_to(n)
        i0 = _pad_idx(idx[0], n_pad)
        i1 = _pad_idx(idx[1], n_pad)
        eps, sig = _ew_call(_pair_kernel, 2, [
            k_atom[i0].reshape(-1, 128), k_atom[i1].reshape(-1, 128),
            eq_atom[i0].reshape(-1, 128), eq_atom[i1].reshape(-1, 128)],
            n_pad)
        return eps[:n], sig[:n]

    eps14, sig14 = pair_terms(one_four_idx)
    epsnb, signb = pair_terms(nonbonded_idx)

    return (k_atom, eq_atom, k_bond, eq_bond, k_angle, eq_angle,
            k_torsion, eq_torsion, u0, eps14, sig14, epsnb, signb)


# speedup vs baseline: 18.2225x; 18.2225x over previous
import functools

import jax
import jax.numpy as jnp
from jax import lax
from jax.experimental import pallas as pl
from jax.experimental.pallas import tpu as pltpu
from jax.experimental.pallas import tpu_sc as plsc

_NA = 50000
_NM = 2000
_DIM = 128
_TW = 128


def _sc_gather(table, idx_all, b_pad, chunk):
    mesh = plsc.VectorSubcoreMesh(core_axis_name="c", subcore_axis_name="s")
    info = plsc.get_sparse_core_info()
    nw = info.num_cores * info.num_subcores
    b_per_w = b_pad // nw
    n_chunks = b_per_w // chunk

    @functools.partial(
        pl.kernel, mesh=mesh,
        out_type=jax.ShapeDtypeStruct((b_pad, _TW), jnp.float32),
        scratch_types=[
            pltpu.VMEM((chunk,), jnp.int32),
            pltpu.VMEM((chunk, _TW), jnp.float32),
            pltpu.SemaphoreType.DMA,
        ],
    )
    def gk(idx_hbm, table_hbm, out_hbm, idx_v, rows_v, sem):
        wid = lax.axis_index("s") * info.num_cores + lax.axis_index("c")
        base = wid * b_per_w

        def body(j, carry):
            off = base + j * chunk
            pltpu.sync_copy(idx_hbm.at[pl.ds(off, chunk)], idx_v)
            pltpu.async_copy(table_hbm.at[idx_v], rows_v, sem).wait()
            pltpu.sync_copy(rows_v, out_hbm.at[pl.ds(off, chunk)])
            return carry

        lax.fori_loop(0, n_chunks, body, 0)

    return gk(idx_all, table)


def _proj_kernel(h_ref, w_ref, b_ref, o_ref):
    o_ref[...] = jnp.dot(h_ref[...], w_ref[...],
                         preferred_element_type=jnp.float32) + b_ref[...]


def _segsum_kernel(ids_ref, h_ref, o_ref):
    i = pl.program_id(0)

    @pl.when(i == 0)
    def _():
        o_ref[...] = jnp.zeros_like(o_ref)

    ids = ids_ref[...]
    mol_iota = jax.lax.broadcasted_iota(jnp.int32, (ids.shape[0], _NM), 1)
    onehot = (ids == mol_iota).astype(jnp.float32)
    o_ref[...] += jax.lax.dot_general(
        onehot, h_ref[...], (((0,), (0,)), ((), ())),
        preferred_element_type=jnp.float32)


def _mol_mlp_kernel(hm_ref, w1_ref, b1_ref, w2_ref, b2_ref, o_ref):
    t = jnp.tanh(jnp.dot(hm_ref[...], w1_ref[...],
                         preferred_element_type=jnp.float32) + b1_ref[...])
    o_ref[...] = jnp.dot(t, w2_ref[...],
                         preferred_element_type=jnp.float32) + b2_ref[...]


def _sum2_kernel(a_ref, b_ref, c_ref, d_ref, ok_ref, oe_ref):
    ok_ref[...] = a_ref[...] + b_ref[...]
    oe_ref[...] = c_ref[...] + d_ref[...]


def _sum3_kernel(a_ref, b_ref, c_ref, d_ref, e_ref, f_ref, ok_ref, oe_ref):
    ok_ref[...] = a_ref[...] + b_ref[...] + c_ref[...]
    oe_ref[...] = d_ref[...] + e_ref[...] + f_ref[...]


def _pair_kernel(k0_ref, k1_ref, e0_ref, e1_ref, oeps_ref, osig_ref):
    oeps_ref[...] = jnp.sqrt(jnp.abs(k0_ref[...] * k1_ref[...]))
    osig_ref[...] = e0_ref[...] * e1_ref[...]


def _pad_rows(x, rows):
    return jnp.concatenate(
        [x, jnp.zeros((rows - x.shape[0],) + x.shape[1:], x.dtype)], axis=0)


def _as_blocks(v, n_pad):
    v = jnp.concatenate([v, jnp.zeros((n_pad - v.shape[0],), v.dtype)])
    return v.reshape(n_pad // 128, 128)


def _pad_idx(idx, n_pad):
    return jnp.concatenate(
        [idx, jnp.zeros((n_pad - idx.shape[0],), idx.dtype)])


def _ew_call(body, n_out, inputs, n_pad):
    rows = n_pad // 128
    outs = pl.pallas_call(
        body,
        out_shape=[jax.ShapeDtypeStruct((rows, 128), jnp.float32)] * n_out,
    )(*inputs)
    return [o.reshape(-1) for o in outs]


def kernel(h, bond_idx, angle_idx0, angle_idx1, angle_idx2, torsion_idx2,
           torsion_idx3, mol_ids, one_four_idx, nonbonded_idx,
           atom_W1, atom_b1, atom_W2, atom_b2,
           bond_W1, bond_b1, bond_W2, bond_b2,
           angle_W1, angle_b1, angle_W2, angle_b2,
           torsion_W1, torsion_b1, torsion_W2, torsion_b2,
           angle0_W, angle0_b, mol_W1, mol_b1, mol_W2, mol_b2):
    f32 = jnp.float32
    d = h.shape[1]

    wc_atom = atom_W1 @ atom_W2
    bc_atom = atom_b1 @ atom_W2 + atom_b2
    wc_bond = bond_W1 @ bond_W2
    bc_bond = bond_b1 @ bond_W2 + bond_b2
    wc_tor = torsion_W1 @ torsion_W2
    bc_tor = torsion_b1 @ torsion_W2 + torsion_b2
    wc_ang = angle_W1 @ angle_W2
    bc_ang = angle_b1 @ angle_W2 + angle_b2
    wa = angle0_W[:d]
    wb = angle0_W[d:2 * d]
    wcq = angle0_W[2 * d:]
    w_ang_ac = (wa + wcq) @ wc_ang
    w_ang_b = 2.0 * (wb @ wc_ang)
    c_ang = 2.0 * (angle0_b @ wc_ang) + bc_ang

    wbig = jnp.zeros((d, _DIM), f32)
    wbig = wbig.at[:, 0:2].set(wc_atom)
    wbig = wbig.at[:, 2:4].set(wc_bond)
    wbig = wbig.at[:, 4:6].set(wc_tor)
    wbig = wbig.at[:, 6:8].set(w_ang_ac)
    wbig = wbig.at[:, 8:10].set(w_ang_b)
    bvec = jnp.zeros((_DIM,), f32)
    bvec = bvec.at[0:2].set(bc_atom)
    bvec = bvec.at[2:4].set(0.5 * bc_bond)
    bvec = bvec.at[4:6].set(0.5 * bc_tor)
    bvec = bvec.at[8:10].set(c_ang)

    blk = 2000
    p = pl.pallas_call(
        _proj_kernel,
        grid=(_NA // blk,),
        in_specs=[pl.BlockSpec((blk, d), lambda i: (i, 0)),
                  pl.BlockSpec((d, _DIM), lambda i: (0, 0)),
                  pl.BlockSpec((1, _DIM), lambda i: (0, 0))],
        out_specs=pl.BlockSpec((blk, _DIM), lambda i: (i, 0)),
        out_shape=jax.ShapeDtypeStruct((_NA, _DIM), f32),
    )(h, wbig, bvec[None, :])

    k_atom = p[:, 0]
    eq_atom = p[:, 1]
    pk_bond = p[:, 2]
    pe_bond = p[:, 3]
    pk_tor = p[:, 4]
    pe_tor = p[:, 5]
    pk_ac = p[:, 6]
    pe_ac = p[:, 7]
    pk_b = p[:, 8]
    pe_b = p[:, 9]

    ba = 512
    na_pad = ((_NA + ba - 1) // ba) * ba
    h_pad = _pad_rows(h, na_pad)
    ids_pad = jnp.concatenate(
        [mol_ids.astype(jnp.int32),
         jnp.full((na_pad - _NA,), -1, jnp.int32)]).reshape(na_pad, 1)
    h_mol = pl.pallas_call(
        _segsum_kernel,
        grid=(na_pad // ba,),
        in_specs=[pl.BlockSpec((ba, 1), lambda i: (i, 0)),
                  pl.BlockSpec((ba, d), lambda i: (i, 0))],
        out_specs=pl.BlockSpec((_NM, d), lambda i: (0, 0)),
        out_shape=jax.ShapeDtypeStruct((_NM, d), f32),
    )(ids_pad, h_pad)

    ru = mol_W1.shape[1]
    w2p = jnp.zeros((ru, _DIM), f32).at[:, 0:1].set(mol_W2)
    b2p = jnp.zeros((_DIM,), f32).at[0:1].set(mol_b2)
    u0 = pl.pallas_call(
        _mol_mlp_kernel,
        out_shape=jax.ShapeDtypeStruct((_NM, _DIM), f32),
    )(h_mol, mol_W1, mol_b1[None, :], w2p, b2p[None, :])[:, 0]

    def pad_to(n):
        return ((n + 1023) // 1024) * 1024

    table = p

    n_bond = bond_idx.shape[1]
    n_tor = torsion_idx2.shape[0]
    n_ang = angle_idx0.shape[0]
    n_14 = one_four_idx.shape[1]
    n_nb = nonbonded_idx.shape[1]
    np_b, np_t, np_g = pad_to(n_bond), pad_to(n_tor), pad_to(n_ang)
    np_f, np_n = pad_to(n_14), pad_to(n_nb)

    segs = [bond_idx[0], bond_idx[1], torsion_idx3, torsion_idx2,
            angle_idx0, angle_idx2, angle_idx1,
            one_four_idx[0], one_four_idx[1],
            nonbonded_idx[0], nonbonded_idx[1]]
    lens = [np_b, np_b, np_t, np_t, np_g, np_g, np_g,
            np_f, np_f, np_n, np_n]
    offs = []
    o = 0
    for ln in lens:
        offs.append(o)
        o += ln
    chunk = 256
    info = plsc.get_sparse_core_info()
    gran = chunk * info.num_cores * info.num_subcores
    b_pad = ((o + gran - 1) // gran) * gran
    idx_all = jnp.concatenate(
        [jnp.concatenate([s.astype(jnp.int32),
                          jnp.zeros((ln - s.shape[0],), jnp.int32)])
         for s, ln in zip(segs, lens)] +
        [jnp.zeros((b_pad - o,), jnp.int32)])

    g = _sc_gather(table, idx_all, b_pad, chunk)

    def col(seg_i, c, ln):
        return g[offs[seg_i]:offs[seg_i] + ln, c].reshape(-1, 128)

    kb, eb = _ew_call(_sum2_kernel, 2, [
        col(0, 2, np_b), col(1, 2, np_b),
        col(0, 3, np_b), col(1, 3, np_b)], np_b)
    k_bond = kb[:n_bond]
    eq_bond = eb[:n_bond]

    kt, et = _ew_call(_sum2_kernel, 2, [
        col(2, 4, np_t), col(3, 4, np_t),
        col(2, 5, np_t), col(3, 5, np_t)], np_t)
    k_torsion = kt[:n_tor]
    eq_torsion = et[:n_tor]

    ka, ea = _ew_call(_sum3_kernel, 2, [
        col(4, 6, np_g), col(5, 6, np_g), col(6, 8, np_g),
        col(4, 7, np_g), col(5, 7, np_g), col(6, 9, np_g)], np_g)
    k_angle = ka[:n_ang]
    eq_angle = ea[:n_ang]

    eps14, sig14 = _ew_call(_pair_kernel, 2, [
        col(7, 0, np_f), col(8, 0, np_f),
        col(7, 1, np_f), col(8, 1, np_f)], np_f)
    eps14 = eps14[:n_14]
    sig14 = sig14[:n_14]

    epsnb, signb = _ew_call(_pair_kernel, 2, [
        col(9, 0, np_n), col(10, 0, np_n),
        col(9, 1, np_n), col(10, 1, np_n)], np_n)
    epsnb = epsnb[:n_nb]
    signb = signb[:n_nb]

    return (k_atom, eq_atom, k_bond, eq_bond, k_angle, eq_angle,
            k_torsion, eq_torsion, u0, eps14, sig14, epsnb, signb)
